# baseline (speedup 1.0000x reference)
import functools

import jax
import jax.numpy as jnp
from jax import lax
from jax.experimental import pallas as pl
from jax.experimental.pallas import tpu as pltpu
from jax.experimental.pallas import tpu_sc as plsc

V = 1000
S = 50
NC = 2
NS = 16
NW = NC * NS
Q = 4
CB = 16

mesh = plsc.VectorSubcoreMesh(core_axis_name="c", subcore_axis_name="s")


@functools.partial(
    pl.kernel,
    mesh=mesh,
    compiler_params=pltpu.CompilerParams(
        use_tc_tiling_on_sc=False, needs_layout_passes=False
    ),
    out_type=jax.ShapeDtypeStruct((S, V // 8, 8, 8, 128), jnp.float32),
    scratch_types=[
        pltpu.VMEM((S, 2 * CB), jnp.int32),
        pltpu.VMEM((CB, V), jnp.float32),
        pltpu.VMEM((CB, V), jnp.float32),
        pltpu.VMEM((V // 8, 8, CB), jnp.float32),
        pltpu.VMEM_SHARED((V, V), jnp.float32),
        pltpu.SemaphoreType.DMA,
        pltpu.SemaphoreType.DMA,
        pltpu.SemaphoreType.DMA,
    ],
)
def _bigram(tokT, table, out, idxbuf, rows0, rows1, tbuf, tabs, semg0, semg1, semw):
    sid = lax.axis_index("s")
    wid = sid * NC + lax.axis_index("c")
    bt = wid // Q
    q = wid % Q
    col0 = bt * 128 + q * 2 * CB

    @pl.when(sid < 8)
    def _():
        pltpu.sync_copy(
            table.at[pl.ds(sid * (V // 8), V // 8)],
            tabs.at[pl.ds(sid * (V // 8), V // 8)],
        )

    pltpu.sync_copy(tokT.at[:, pl.ds(col0, 2 * CB)], idxbuf)
    plsc.subcore_barrier()

    def idx_ref(k):
        return idxbuf.at[k // 2, pl.ds((k % 2) * CB, CB)]

    def gather_start(k, rows, semg):
        pltpu.async_copy(tabs.at[idx_ref(k)], rows, semg)

    def gather_wait(k, rows, semg):
        pltpu.make_async_copy(tabs.at[idx_ref(k)], rows, semg).wait()

    def out_ref(k):
        return out.at[k // 2, :, bt, :, pl.ds(q * 2 * CB + (k % 2) * CB, CB)]

    iota = lax.iota(jnp.int32, 16)

    def transpose(rows):
        @plsc.parallel_loop(0, V // 8, unroll=2)
        def tbody(vt):
            for vi in range(8):
                colv = jnp.zeros((16,), jnp.int32) + (vt * 8 + vi)
                vals = plsc.load_gather(rows, [iota, colv])
                tbuf[vt, vi, pl.ds(0, CB)] = vals

    NT = 2 * S
    gather_start(0, rows0, semg0)

    def body(j, _):
        for b, (cur, semc, nxt, semn) in enumerate(
            ((rows0, semg0, rows1, semg1), (rows1, semg1, rows0, semg0))
        ):
            k = j * 2 + b
            gather_wait(k, cur, semc)

            @pl.when(k < NT - 1)
            def _():
                gather_start(k + 1, nxt, semn)

            @pl.when(k > 0)
            def _():
                pltpu.make_async_copy(tbuf, out_ref(jnp.maximum(k - 1, 0)), semw).wait()

            transpose(cur)
            pltpu.async_copy(tbuf, out_ref(k), semw)
        return 0

    lax.fori_loop(0, NT // 2, body, 0)
    pltpu.make_async_copy(tbuf, out_ref(NT - 1), semw).wait()


def kernel(tokens, lookup_table):
    b, s = tokens.shape
    tokT = tokens.T.astype(jnp.int32)
    x5 = _bigram(tokT, lookup_table)
    return jnp.transpose(x5, (2, 4, 0, 1, 3)).reshape(b, s, V)
